# baseline (speedup 1.0000x reference)
import functools

import jax
import jax.numpy as jnp
from jax import lax
from jax.experimental import pallas as pl
from jax.experimental.pallas import tpu as pltpu
from jax.experimental.pallas import tpu_sc as plsc

_B, _S = 4, 8192
_V, _D = 1024, 128
_TOT = _B * _S

_info = plsc.get_sparse_core_info()
_NC, _NS = _info.num_cores, _info.num_subcores
_NW = _NC * _NS
_B_PER_W = _TOT // _NW
_CH = 128
_NCHUNK = _B_PER_W // _CH


def _pos_body(ids_ref, pos_ref):
    ids = ids_ref[...]
    dm_b = (ids == 1) | ((ids >= 17) & (ids <= 26))
    om = dm_b | (ids == 12) | (ids == 30)
    dm = dm_b.astype(jnp.int32)
    idx = lax.broadcasted_iota(jnp.int32, ids.shape, 1)
    prev = jnp.concatenate(
        [jnp.zeros((ids.shape[0], 1), jnp.int32), dm[:, :-1]], axis=1)
    starts = dm * (1 - prev)
    v = idx * starts
    k = 1
    while k < ids.shape[1]:
        shifted = jnp.concatenate(
            [jnp.zeros((ids.shape[0], k), jnp.int32), v[:, :-k]], axis=1)
        v = jnp.maximum(v, shifted)
        k *= 2
    pos = (idx - v + 1) * om.astype(jnp.int32)
    pos_ref[...] = jnp.clip(pos, 0, _V - 1)


_positions = pl.pallas_call(
    _pos_body,
    out_shape=jax.ShapeDtypeStruct((_B, _S), jnp.int32),
)


def _gather_body(table_hbm, idx_hbm, out_hbm, idx_v, buf0, buf1, gsem):
    wid = lax.axis_index("s") * _NC + lax.axis_index("c")
    base = wid * _B_PER_W
    pltpu.sync_copy(idx_hbm.at[wid], idx_v)
    bufs = (buf0, buf1)
    for j in range(_NCHUNK):
        pltpu.async_copy(table_hbm.at[idx_v.at[j]], bufs[j % 2], gsem).wait()
        pltpu.sync_copy(bufs[j % 2], out_hbm.at[pl.ds(base + j * _CH, _CH)])


_gather = functools.partial(
    pl.kernel,
    mesh=plsc.VectorSubcoreMesh(core_axis_name="c", subcore_axis_name="s"),
    out_type=jax.ShapeDtypeStruct((_TOT, _D), jnp.float32),
    scratch_types=[
        pltpu.VMEM((_NCHUNK, _CH), jnp.int32),
        pltpu.VMEM((_CH, _D), jnp.float32),
        pltpu.VMEM((_CH, _D), jnp.float32),
        pltpu.SemaphoreType.DMA,
    ],
)(_gather_body)


def kernel(input_ids, embedding):
    pos = _positions(input_ids)
    pos = pos.reshape(_NW, _NCHUNK, _CH)
    out = _gather(embedding, pos)
    return out.reshape(_B, _S, _D)


# speedup vs baseline: 12.7863x; 12.7863x over previous
import functools

import jax
import jax.numpy as jnp
from jax import lax
from jax.experimental import pallas as pl
from jax.experimental.pallas import tpu as pltpu
from jax.experimental.pallas import tpu_sc as plsc

_B, _S = 4, 8192
_V, _D = 1024, 128
_TOT = _B * _S

_info = plsc.get_sparse_core_info()
_NC, _NS = _info.num_cores, _info.num_subcores
_NW = _NC * _NS
_NPAIR = _NW // 2
_ROWS_PER_PAIR = _TOT // _NPAIR
_HD = _D // 2
_CH = 128
_NCHUNK = _ROWS_PER_PAIR // _CH


def _pos_body(ids_ref, pos_ref):
    ids = ids_ref[...]
    dm_b = (ids == 1) | ((ids >= 17) & (ids <= 26))
    om = dm_b | (ids == 12) | (ids == 30)
    dm = dm_b.astype(jnp.int32)
    idx = lax.broadcasted_iota(jnp.int32, ids.shape, 1)
    prev = jnp.concatenate(
        [jnp.zeros((ids.shape[0], 1), jnp.int32), dm[:, :-1]], axis=1)
    starts = dm * (1 - prev)
    v = idx * starts
    k = 1
    while k < ids.shape[1]:
        shifted = jnp.concatenate(
            [jnp.zeros((ids.shape[0], k), jnp.int32), v[:, :-k]], axis=1)
        v = jnp.maximum(v, shifted)
        k *= 2
    pos = (idx - v + 1) * om.astype(jnp.int32)
    pos_ref[...] = jnp.clip(pos, 0, _V - 1)


_positions = pl.pallas_call(
    _pos_body,
    out_shape=jax.ShapeDtypeStruct((_B, _S), jnp.int32),
)


def _gather_body(table_hbm, idx_hbm, out_hbm, idx_v, table_v, buf0, buf1,
                 lsem, wsem):
    wid = lax.axis_index("s") * _NC + lax.axis_index("c")
    pair = wid // 2
    half = wid % 2
    base = pair * _ROWS_PER_PAIR
    col0 = half * _HD

    icopy = pltpu.async_copy(idx_hbm.at[pair], idx_v, lsem)
    tcopy = pltpu.async_copy(
        table_hbm.at[:, pl.ds(col0, _HD)], table_v, lsem)
    icopy.wait()
    tcopy.wait()

    bufs = (buf0, buf1)
    wcopies = [None] * _NCHUNK
    for c in range(_NCHUNK):
        if c >= 2:
            wcopies[c - 2].wait()
        buf = bufs[c % 2]

        def row_body(g, _, c=c, buf=buf):
            posv = idx_v[c, pl.ds(g * 16, 16)]
            for j in range(16):
                r = g * 16 + j
                row = posv[j]
                for k in range(_HD // 16):
                    buf[r, pl.ds(k * 16, 16)] = table_v[row, pl.ds(k * 16, 16)]
            return 0

        lax.fori_loop(0, _CH // 16, row_body, 0, unroll=False)
        wcopies[c] = pltpu.async_copy(
            buf,
            out_hbm.at[pl.ds(base + c * _CH, _CH), pl.ds(col0, _HD)],
            wsem)
    wcopies[_NCHUNK - 2].wait()
    wcopies[_NCHUNK - 1].wait()


_gather = functools.partial(
    pl.kernel,
    mesh=plsc.VectorSubcoreMesh(core_axis_name="c", subcore_axis_name="s"),
    out_type=jax.ShapeDtypeStruct((_TOT, _D), jnp.float32),
    scratch_types=[
        pltpu.VMEM((_NCHUNK, _CH), jnp.int32),
        pltpu.VMEM((_V, _HD), jnp.float32),
        pltpu.VMEM((_CH, _HD), jnp.float32),
        pltpu.VMEM((_CH, _HD), jnp.float32),
        pltpu.SemaphoreType.DMA,
        pltpu.SemaphoreType.DMA,
    ],
    compiler_params=pltpu.CompilerParams(use_tc_tiling_on_sc=False),
)(_gather_body)


def kernel(input_ids, embedding):
    pos = _positions(input_ids)
    pos = pos.reshape(_NPAIR, _NCHUNK, _CH)
    out = _gather(embedding, pos)
    return out.reshape(_B, _S, _D)
